# Initial kernel scaffold; baseline (speedup 1.0000x reference)
import jax
import jax.numpy as jnp
from jax.experimental import pallas as pl


def kernel(x, edge_index, edge_weight, W1, b1, W2, b2):
    raise NotImplementedError("write your pallas kernel here")


# speedup vs baseline: 21.2706x; 21.2706x over previous
import functools

import jax
import jax.numpy as jnp
from jax import lax
from jax.experimental import pallas as pl
from jax.experimental.pallas import tpu as pltpu
from jax.experimental.pallas import tpu_sc as plsc

N = 10000
D = 128
H = 32
C = 40
CP = 48
K = 10
ALPHA = 0.1

NC = 2
NS = 16
NW = NC * NS
BS = 128
NB = 82
PT = NB * BS
ET_PAD = NW * PT
NP = 10240
NSL = NP // NS
ZR = 128

_MESH = plsc.VectorSubcoreMesh(
    core_axis_name="c", subcore_axis_name="s", num_cores=NC, num_subcores=NS
)
_SC_PARAMS = pltpu.CompilerParams(use_tc_tiling_on_sc=False, needs_layout_passes=False)


def _mlp_body(x_ref, w1_ref, b1_ref, w2_ref, b2_ref, o_ref):
    a = jnp.dot(x_ref[...], w1_ref[...], preferred_element_type=jnp.float32)
    a = jnp.maximum(a + b1_ref[...], 0.0)
    o_ref[...] = (
        jnp.dot(a, w2_ref[...], preferred_element_type=jnp.float32) + b2_ref[...]
    )


def _mlp(x, W1, b1r, W2p, b2p):
    blk = 1024
    return pl.pallas_call(
        _mlp_body,
        grid=(NP // blk,),
        in_specs=[
            pl.BlockSpec((blk, D), lambda i: (i, 0)),
            pl.BlockSpec((D, H), lambda i: (0, 0)),
            pl.BlockSpec((1, H), lambda i: (0, 0)),
            pl.BlockSpec((H, CP), lambda i: (0, 0)),
            pl.BlockSpec((1, CP), lambda i: (0, 0)),
        ],
        out_specs=pl.BlockSpec((blk, CP), lambda i: (i, 0)),
        out_shape=jax.ShapeDtypeStruct((NP, CP), jnp.float32),
    )(x, W1, b1r, W2p, b2p)


@functools.partial(
    pl.kernel,
    out_type=jax.ShapeDtypeStruct((NC * NP, 16), jnp.float32),
    mesh=_MESH,
    compiler_params=_SC_PARAMS,
    scratch_types=[
        pltpu.VMEM((NB, BS), jnp.int32),
        pltpu.VMEM((NB, BS), jnp.float32),
        pltpu.VMEM((BS, 16), jnp.float32),
        pltpu.VMEM((ZR, 16), jnp.float32),
        pltpu.VMEM((NSL, 16), jnp.float32),
        pltpu.VMEM_SHARED((NP, 16), jnp.float32),
    ],
)
def _deg_kernel(colg, wg, degp, col_v, w_v, srows, zbuf, obuf, deg_sh):
    cid = lax.axis_index("c")
    sid = lax.axis_index("s")
    wid = cid * NS + sid
    pltpu.sync_copy(colg.at[wid], col_v)
    pltpu.sync_copy(wg.at[wid], w_v)

    @pl.loop(0, ZR)
    def _z(i):
        zbuf[i, :] = jnp.zeros((16,), jnp.float32)

    @pl.loop(0, NSL // ZR)
    def _zc(j):
        pltpu.sync_copy(zbuf, deg_sh.at[pl.ds(sid * NSL + j * ZR, ZR)])

    plsc.subcore_barrier()

    @pl.loop(0, NB)
    def _b(b):
        @pl.loop(0, BS // 16)
        def _g(g):
            w16 = w_v[b, pl.ds(g * 16, 16)]
            for l in range(16):
                srows[g * 16 + l, :] = jnp.full((16,), w16[l], jnp.float32)

        pltpu.sync_copy(srows, deg_sh.at[col_v.at[b]], add=True)

    plsc.subcore_barrier()
    pltpu.sync_copy(deg_sh.at[pl.ds(sid * NSL, NSL)], obuf)
    pltpu.sync_copy(obuf, degp.at[pl.ds(cid * NP + sid * NSL, NSL)])


def _dinv_body(degp_ref, o_ref):
    d = jnp.sum(degp_ref[0] + degp_ref[1], axis=1) * (1.0 / 16.0)
    o_ref[...] = jnp.where(d > 0, lax.rsqrt(jnp.where(d > 0, d, 1.0)), 0.0)


def _dinv(degp):
    return pl.pallas_call(
        _dinv_body,
        in_specs=[pl.BlockSpec((NC, NP, 16), lambda: (0, 0, 0))],
        out_specs=pl.BlockSpec((NP,), lambda: (0,)),
        out_shape=jax.ShapeDtypeStruct((NP,), jnp.float32),
    )(degp)


@functools.partial(
    pl.kernel,
    out_type=jax.ShapeDtypeStruct((NW, NB, BS), jnp.float32),
    mesh=_MESH,
    compiler_params=_SC_PARAMS,
    scratch_types=[
        pltpu.VMEM((NP,), jnp.float32),
        pltpu.VMEM((NB, BS), jnp.int32),
        pltpu.VMEM((NB, BS), jnp.int32),
        pltpu.VMEM((NB, BS), jnp.float32),
        pltpu.VMEM((NB, BS), jnp.float32),
    ],
)
def _norm_kernel(dinv, rowg, colg, wg, normg, dinv_v, row_v, col_v, w_v, norm_v):
    cid = lax.axis_index("c")
    sid = lax.axis_index("s")
    wid = cid * NS + sid
    pltpu.sync_copy(dinv, dinv_v)
    pltpu.sync_copy(rowg.at[wid], row_v)
    pltpu.sync_copy(colg.at[wid], col_v)
    pltpu.sync_copy(wg.at[wid], w_v)

    @pl.loop(0, NB)
    def _b(b):
        @pl.loop(0, BS // 16)
        def _g(g):
            sl = pl.ds(g * 16, 16)
            r16 = row_v[b, sl]
            c16 = col_v[b, sl]
            w16 = w_v[b, sl]
            dr = plsc.load_gather(dinv_v, [r16])
            dc = plsc.load_gather(dinv_v, [c16])
            norm_v[b, sl] = dr * w16 * dc

    pltpu.sync_copy(norm_v, normg.at[wid])


@functools.partial(
    pl.kernel,
    out_type=jax.ShapeDtypeStruct((NC * NP, CP), jnp.float32),
    mesh=_MESH,
    compiler_params=_SC_PARAMS,
    scratch_types=[
        pltpu.VMEM((NB, BS), jnp.int32),
        pltpu.VMEM((NB, BS), jnp.int32),
        pltpu.VMEM((NB, BS), jnp.float32),
        pltpu.VMEM((BS, CP), jnp.float32),
        pltpu.VMEM((BS, CP), jnp.float32),
        pltpu.VMEM((ZR, CP), jnp.float32),
        pltpu.VMEM((NSL, CP), jnp.float32),
        pltpu.VMEM_SHARED((NP, CP), jnp.float32),
        pltpu.SemaphoreType.DMA,
        pltpu.SemaphoreType.DMA,
    ],
)
def _iter_kernel(
    out_hbm, rowg, colg, normg, p_out,
    row_v, col_v, norm_v, rows0, rows1, zbuf, obuf, agg_sh, gsem0, gsem1,
):
    cid = lax.axis_index("c")
    sid = lax.axis_index("s")
    wid = cid * NS + sid
    pltpu.sync_copy(rowg.at[wid], row_v)
    pltpu.sync_copy(colg.at[wid], col_v)
    pltpu.sync_copy(normg.at[wid], norm_v)

    @pl.loop(0, ZR)
    def _z(i):
        for k in range(CP // 16):
            zbuf[i, pl.ds(k * 16, 16)] = jnp.zeros((16,), jnp.float32)

    @pl.loop(0, NSL // ZR)
    def _zc(j):
        pltpu.sync_copy(zbuf, agg_sh.at[pl.ds(sid * NSL + j * ZR, ZR)])

    plsc.subcore_barrier()

    def _process(b, rows_v):
        @pl.loop(0, BS // 16)
        def _g(g):
            n16 = norm_v[b, pl.ds(g * 16, 16)]
            for l in range(16):
                e = g * 16 + l
                nbv = jnp.full((16,), n16[l], jnp.float32)
                for k in range(CP // 16):
                    sl = pl.ds(k * 16, 16)
                    rows_v[e, sl] = rows_v[e, sl] * nbv

        pltpu.sync_copy(rows_v, agg_sh.at[col_v.at[b]], add=True)

    pltpu.async_copy(out_hbm.at[row_v.at[0]], rows0, gsem0)

    @pl.loop(0, NB, step=2)
    def _b(b0):
        pltpu.make_async_copy(out_hbm.at[row_v.at[b0]], rows0, gsem0).wait()
        pltpu.async_copy(out_hbm.at[row_v.at[b0 + 1]], rows1, gsem1)
        _process(b0, rows0)
        pltpu.make_async_copy(out_hbm.at[row_v.at[b0 + 1]], rows1, gsem1).wait()

        @pl.when(b0 + 2 < NB)
        def _():
            pltpu.async_copy(out_hbm.at[row_v.at[b0 + 2]], rows0, gsem0)

        _process(b0 + 1, rows1)

    plsc.subcore_barrier()
    pltpu.sync_copy(agg_sh.at[pl.ds(sid * NSL, NSL)], obuf)
    pltpu.sync_copy(obuf, p_out.at[pl.ds(cid * NP + sid * NSL, NSL)])


def _comb_body(p_ref, h_ref, o_ref):
    pb = p_ref[...]
    o_ref[...] = (1.0 - ALPHA) * (pb[0] + pb[1]) + ALPHA * h_ref[...]


def _combine(p, h):
    blk = 1024
    return pl.pallas_call(
        _comb_body,
        grid=(NP // blk,),
        in_specs=[
            pl.BlockSpec((NC, blk, CP), lambda i: (0, i, 0)),
            pl.BlockSpec((blk, CP), lambda i: (i, 0)),
        ],
        out_specs=pl.BlockSpec((blk, CP), lambda i: (i, 0)),
        out_shape=jax.ShapeDtypeStruct((NP, CP), jnp.float32),
    )(p, h)


def _final_body(p_ref, h_ref, o_ref):
    pb = p_ref[...]
    o = (1.0 - ALPHA) * (pb[0] + pb[1]) + ALPHA * h_ref[...]
    colid = lax.broadcasted_iota(jnp.int32, o.shape, 1)
    valid = colid < C
    om = jnp.where(valid, o, jnp.float32(-1e30))
    m = jnp.max(om, axis=1, keepdims=True)
    ex = jnp.where(valid, jnp.exp(o - m), 0.0)
    s = jnp.sum(ex, axis=1, keepdims=True)
    o_ref[...] = o - m - jnp.log(s)


def _final(p, h):
    blk = 1024
    return pl.pallas_call(
        _final_body,
        grid=(NP // blk,),
        in_specs=[
            pl.BlockSpec((NC, blk, CP), lambda i: (0, i, 0)),
            pl.BlockSpec((blk, CP), lambda i: (i, 0)),
        ],
        out_specs=pl.BlockSpec((blk, CP), lambda i: (i, 0)),
        out_shape=jax.ShapeDtypeStruct((NP, CP), jnp.float32),
    )(p, h)


def kernel(x, edge_index, edge_weight, W1, b1, W2, b2):
    row = edge_index[0].astype(jnp.int32)
    col = edge_index[1].astype(jnp.int32)
    loop = jnp.arange(N, dtype=jnp.int32)
    row = jnp.concatenate([row, loop])
    col = jnp.concatenate([col, loop])
    w = jnp.concatenate([edge_weight.astype(jnp.float32), jnp.ones((N,), jnp.float32)])

    pad = ET_PAD - row.shape[0]
    pidx = (jnp.arange(pad, dtype=jnp.int32) * 31) % N
    row_g = jnp.concatenate([row, pidx]).reshape(NW, NB, BS)
    col_g = jnp.concatenate([col, pidx]).reshape(NW, NB, BS)
    w_g = jnp.concatenate([w, jnp.zeros((pad,), jnp.float32)]).reshape(NW, NB, BS)

    W2p = jnp.pad(W2, ((0, 0), (0, CP - C)))
    b2p = jnp.pad(b2, (0, CP - C)).reshape(1, CP)
    b1r = b1.reshape(1, H)

    xp = jnp.pad(x, ((0, NP - N), (0, 0)))
    h = _mlp(xp, W1, b1r, W2p, b2p)
    degp = _deg_kernel(col_g, w_g).reshape(NC, NP, 16)
    dinv = _dinv(degp)
    norm_g = _norm_kernel(dinv, row_g, col_g, w_g)

    out = h
    for _ in range(K - 1):
        p = _iter_kernel(out, row_g, col_g, norm_g).reshape(NC, NP, CP)
        out = _combine(p, h)
    p = _iter_kernel(out, row_g, col_g, norm_g).reshape(NC, NP, CP)
    res = _final(p, h)
    return res[:N, :C]


# speedup vs baseline: 27.1875x; 1.2782x over previous
import functools

import jax
import jax.numpy as jnp
from jax import lax
from jax.experimental import pallas as pl
from jax.experimental.pallas import tpu as pltpu
from jax.experimental.pallas import tpu_sc as plsc

N = 10000
D = 128
H = 32
C = 40
CP = 48
K = 10
ALPHA = 0.1

NC = 2
NS = 16
NW = NC * NS
BS = 128
NB = 84
GB = 3
NBB = NB // GB
BSB = GB * BS
PT = NB * BS
ET_PAD = NW * PT
NP = 10240
NSL = NP // NS
ZR = 128

_MESH = plsc.VectorSubcoreMesh(
    core_axis_name="c", subcore_axis_name="s", num_cores=NC, num_subcores=NS
)
_SC_PARAMS = pltpu.CompilerParams(use_tc_tiling_on_sc=False, needs_layout_passes=False)


def _mlp_body(x_ref, w1_ref, b1_ref, w2_ref, b2_ref, o_ref):
    a = jnp.dot(x_ref[...], w1_ref[...], preferred_element_type=jnp.float32)
    a = jnp.maximum(a + b1_ref[...], 0.0)
    o_ref[...] = (
        jnp.dot(a, w2_ref[...], preferred_element_type=jnp.float32) + b2_ref[...]
    )


def _mlp(x, W1, b1r, W2p, b2p):
    blk = 1024
    return pl.pallas_call(
        _mlp_body,
        grid=(NP // blk,),
        in_specs=[
            pl.BlockSpec((blk, D), lambda i: (i, 0)),
            pl.BlockSpec((D, H), lambda i: (0, 0)),
            pl.BlockSpec((1, H), lambda i: (0, 0)),
            pl.BlockSpec((H, CP), lambda i: (0, 0)),
            pl.BlockSpec((1, CP), lambda i: (0, 0)),
        ],
        out_specs=pl.BlockSpec((blk, CP), lambda i: (i, 0)),
        out_shape=jax.ShapeDtypeStruct((NP, CP), jnp.float32),
    )(x, W1, b1r, W2p, b2p)


@functools.partial(
    pl.kernel,
    out_type=jax.ShapeDtypeStruct((NC * NP, 16), jnp.float32),
    mesh=_MESH,
    compiler_params=_SC_PARAMS,
    scratch_types=[
        pltpu.VMEM((NB, BS), jnp.int32),
        pltpu.VMEM((NB, BS), jnp.float32),
        pltpu.VMEM((BS, 16), jnp.float32),
        pltpu.VMEM((ZR, 16), jnp.float32),
        pltpu.VMEM((NSL, 16), jnp.float32),
        pltpu.VMEM_SHARED((NP, 16), jnp.float32),
    ],
)
def _deg_kernel(colg, wg, degp, col_v, w_v, srows, zbuf, obuf, deg_sh):
    cid = lax.axis_index("c")
    sid = lax.axis_index("s")
    wid = cid * NS + sid
    pltpu.sync_copy(colg.at[wid], col_v)
    pltpu.sync_copy(wg.at[wid], w_v)

    @pl.loop(0, ZR)
    def _z(i):
        zbuf[i, :] = jnp.zeros((16,), jnp.float32)

    @pl.loop(0, NSL // ZR)
    def _zc(j):
        pltpu.sync_copy(zbuf, deg_sh.at[pl.ds(sid * NSL + j * ZR, ZR)])

    plsc.subcore_barrier()

    @pl.loop(0, NB)
    def _b(b):
        @pl.loop(0, BS // 16)
        def _g(g):
            w16 = w_v[b, pl.ds(g * 16, 16)]
            for l in range(16):
                srows[g * 16 + l, :] = jnp.full((16,), w16[l], jnp.float32)

        pltpu.sync_copy(srows, deg_sh.at[col_v.at[b]], add=True)

    plsc.subcore_barrier()
    pltpu.sync_copy(deg_sh.at[pl.ds(sid * NSL, NSL)], obuf)
    pltpu.sync_copy(obuf, degp.at[pl.ds(cid * NP + sid * NSL, NSL)])


def _dinv_body(degp_ref, o_ref):
    d = jnp.sum(degp_ref[0] + degp_ref[1], axis=1) * (1.0 / 16.0)
    o_ref[...] = jnp.where(d > 0, lax.rsqrt(jnp.where(d > 0, d, 1.0)), 0.0)


def _dinv(degp):
    return pl.pallas_call(
        _dinv_body,
        in_specs=[pl.BlockSpec((NC, NP, 16), lambda: (0, 0, 0))],
        out_specs=pl.BlockSpec((NP,), lambda: (0,)),
        out_shape=jax.ShapeDtypeStruct((NP,), jnp.float32),
    )(degp)


@functools.partial(
    pl.kernel,
    out_type=jax.ShapeDtypeStruct((NW, NB, BS), jnp.float32),
    mesh=_MESH,
    compiler_params=_SC_PARAMS,
    scratch_types=[
        pltpu.VMEM((NP,), jnp.float32),
        pltpu.VMEM((NB, BS), jnp.int32),
        pltpu.VMEM((NB, BS), jnp.int32),
        pltpu.VMEM((NB, BS), jnp.float32),
        pltpu.VMEM((NB, BS), jnp.float32),
    ],
)
def _norm_kernel(dinv, rowg, colg, wg, normg, dinv_v, row_v, col_v, w_v, norm_v):
    cid = lax.axis_index("c")
    sid = lax.axis_index("s")
    wid = cid * NS + sid
    pltpu.sync_copy(dinv, dinv_v)
    pltpu.sync_copy(rowg.at[wid], row_v)
    pltpu.sync_copy(colg.at[wid], col_v)
    pltpu.sync_copy(wg.at[wid], w_v)

    @pl.loop(0, NB)
    def _b(b):
        @pl.loop(0, BS // 16)
        def _g(g):
            sl = pl.ds(g * 16, 16)
            r16 = row_v[b, sl]
            c16 = col_v[b, sl]
            w16 = w_v[b, sl]
            dr = plsc.load_gather(dinv_v, [r16])
            dc = plsc.load_gather(dinv_v, [c16])
            norm_v[b, sl] = dr * w16 * dc

    pltpu.sync_copy(norm_v, normg.at[wid])


@functools.partial(
    pl.kernel,
    out_type=jax.ShapeDtypeStruct((NC * NP, CP), jnp.float32),
    mesh=_MESH,
    compiler_params=_SC_PARAMS,
    scratch_types=[
        pltpu.VMEM((NB, BS), jnp.int32),
        pltpu.VMEM((NB, BS), jnp.int32),
        pltpu.VMEM((NB, BS), jnp.float32),
        pltpu.VMEM((BSB, CP), jnp.float32),
        pltpu.VMEM((BSB, CP), jnp.float32),
        pltpu.VMEM((ZR, CP), jnp.float32),
        pltpu.VMEM((NSL // 2, CP), jnp.float32),
        pltpu.VMEM_SHARED((NP, CP), jnp.float32),
        pltpu.SemaphoreType.DMA,
        pltpu.SemaphoreType.DMA,
        pltpu.SemaphoreType.DMA,
        pltpu.SemaphoreType.DMA,
    ],
)
def _iter_kernel(
    out_hbm, rowg, colg, normg, p_out,
    row_v, col_v, norm_v, rows0, rows1, zbuf, obuf, agg_sh,
    gsem0, gsem1, ssem0, ssem1,
):
    cid = lax.axis_index("c")
    sid = lax.axis_index("s")
    wid = cid * NS + sid
    pltpu.sync_copy(rowg.at[wid], row_v)
    pltpu.sync_copy(colg.at[wid], col_v)
    pltpu.sync_copy(normg.at[wid], norm_v)

    @pl.loop(0, ZR)
    def _z(i):
        for k in range(CP // 16):
            zbuf[i, pl.ds(k * 16, 16)] = jnp.zeros((16,), jnp.float32)

    @pl.loop(0, NSL // ZR)
    def _zc(j):
        pltpu.sync_copy(zbuf, agg_sh.at[pl.ds(sid * NSL + j * ZR, ZR)])

    plsc.subcore_barrier()

    def _fire_gather(b, rows_v, sem):
        for j in range(GB):
            pltpu.async_copy(
                out_hbm.at[row_v.at[GB * b + j]],
                rows_v.at[pl.ds(j * BS, BS)],
                sem,
            )

    def _drain_gather(b, rows_v, sem):
        for j in range(GB):
            pltpu.make_async_copy(
                out_hbm.at[row_v.at[GB * b + j]],
                rows_v.at[pl.ds(j * BS, BS)],
                sem,
            ).wait()

    def _fire_scatter(b, rows_v, sem):
        for j in range(GB):
            pltpu.async_copy(
                rows_v.at[pl.ds(j * BS, BS)],
                agg_sh.at[col_v.at[GB * b + j]],
                sem,
                add=True,
            )

    def _drain_scatter(b, rows_v, sem):
        for j in range(GB):
            pltpu.make_async_copy(
                rows_v.at[pl.ds(j * BS, BS)],
                agg_sh.at[col_v.at[GB * b + j]],
                sem,
            ).wait()

    def _scale(b, rows_v):
        @pl.loop(0, GB)
        def _j(j):
            @pl.loop(0, BS // 16)
            def _g(g):
                n16 = norm_v[GB * b + j, pl.ds(g * 16, 16)]
                for l in range(16):
                    e = j * BS + g * 16 + l
                    nbv = jnp.full((16,), n16[l], jnp.float32)
                    for k in range(CP // 16):
                        sl = pl.ds(k * 16, 16)
                        rows_v[e, sl] = rows_v[e, sl] * nbv

    _fire_gather(0, rows0, gsem0)

    @pl.loop(0, NBB, step=2)
    def _b(b0):
        _drain_gather(b0, rows0, gsem0)

        @pl.when(b0 > 0)
        def _():
            _drain_scatter(b0 - 1, rows1, ssem1)

        _fire_gather(b0 + 1, rows1, gsem1)
        _scale(b0, rows0)
        _fire_scatter(b0, rows0, ssem0)

        _drain_gather(b0 + 1, rows1, gsem1)
        _drain_scatter(b0, rows0, ssem0)

        @pl.when(b0 + 2 < NBB)
        def _():
            _fire_gather(b0 + 2, rows0, gsem0)

        _scale(b0 + 1, rows1)
        _fire_scatter(b0 + 1, rows1, ssem1)

    _drain_scatter(NBB - 1, rows1, ssem1)
    plsc.subcore_barrier()

    @pl.loop(0, 2)
    def _o(c):
        off = sid * NSL + c * (NSL // 2)
        pltpu.sync_copy(agg_sh.at[pl.ds(off, NSL // 2)], obuf)
        pltpu.sync_copy(obuf, p_out.at[pl.ds(cid * NP + off, NSL // 2)])


def _comb_body(p_ref, h_ref, o_ref):
    pb = p_ref[...]
    o_ref[...] = (1.0 - ALPHA) * (pb[0] + pb[1]) + ALPHA * h_ref[...]


def _combine(p, h):
    blk = 1024
    return pl.pallas_call(
        _comb_body,
        grid=(NP // blk,),
        in_specs=[
            pl.BlockSpec((NC, blk, CP), lambda i: (0, i, 0)),
            pl.BlockSpec((blk, CP), lambda i: (i, 0)),
        ],
        out_specs=pl.BlockSpec((blk, CP), lambda i: (i, 0)),
        out_shape=jax.ShapeDtypeStruct((NP, CP), jnp.float32),
    )(p, h)


def _final_body(p_ref, h_ref, o_ref):
    pb = p_ref[...]
    o = (1.0 - ALPHA) * (pb[0] + pb[1]) + ALPHA * h_ref[...]
    colid = lax.broadcasted_iota(jnp.int32, o.shape, 1)
    valid = colid < C
    om = jnp.where(valid, o, jnp.float32(-1e30))
    m = jnp.max(om, axis=1, keepdims=True)
    ex = jnp.where(valid, jnp.exp(o - m), 0.0)
    s = jnp.sum(ex, axis=1, keepdims=True)
    o_ref[...] = o - m - jnp.log(s)


def _final(p, h):
    blk = 1024
    return pl.pallas_call(
        _final_body,
        grid=(NP // blk,),
        in_specs=[
            pl.BlockSpec((NC, blk, CP), lambda i: (0, i, 0)),
            pl.BlockSpec((blk, CP), lambda i: (i, 0)),
        ],
        out_specs=pl.BlockSpec((blk, CP), lambda i: (i, 0)),
        out_shape=jax.ShapeDtypeStruct((NP, CP), jnp.float32),
    )(p, h)


def kernel(x, edge_index, edge_weight, W1, b1, W2, b2):
    row = edge_index[0].astype(jnp.int32)
    col = edge_index[1].astype(jnp.int32)
    loop = jnp.arange(N, dtype=jnp.int32)
    row = jnp.concatenate([row, loop])
    col = jnp.concatenate([col, loop])
    w = jnp.concatenate([edge_weight.astype(jnp.float32), jnp.ones((N,), jnp.float32)])

    pad = ET_PAD - row.shape[0]
    pidx = (jnp.arange(pad, dtype=jnp.int32) * 31) % N
    row_g = jnp.concatenate([row, pidx]).reshape(NW, NB, BS)
    col_g = jnp.concatenate([col, pidx]).reshape(NW, NB, BS)
    w_g = jnp.concatenate([w, jnp.zeros((pad,), jnp.float32)]).reshape(NW, NB, BS)

    W2p = jnp.pad(W2, ((0, 0), (0, CP - C)))
    b2p = jnp.pad(b2, (0, CP - C)).reshape(1, CP)
    b1r = b1.reshape(1, H)

    xp = jnp.pad(x, ((0, NP - N), (0, 0)))
    h = _mlp(xp, W1, b1r, W2p, b2p)
    degp = _deg_kernel(col_g, w_g).reshape(NC, NP, 16)
    dinv = _dinv(degp)
    norm_g = _norm_kernel(dinv, row_g, col_g, w_g)

    out = h
    for _ in range(K - 1):
        p = _iter_kernel(out, row_g, col_g, norm_g).reshape(NC, NP, CP)
        out = _combine(p, h)
    p = _iter_kernel(out, row_g, col_g, norm_g).reshape(NC, NP, CP)
    res = _final(p, h)
    return res[:N, :C]


# speedup vs baseline: 31.7018x; 1.1660x over previous
import functools

import jax
import jax.numpy as jnp
from jax import lax
from jax.experimental import pallas as pl
from jax.experimental.pallas import tpu as pltpu
from jax.experimental.pallas import tpu_sc as plsc

N = 10000
D = 128
H = 32
C = 40
CP = 48
K = 10
ALPHA = 0.1

NC = 2
NS = 16
NW = NC * NS
BS = 128
NB = 84
GB = 3
NBB = NB // GB
BSB = GB * BS
PT = NB * BS
ET_PAD = NW * PT
NP = 10240
NSL = NP // NS
ZR = 128

_MESH = plsc.VectorSubcoreMesh(
    core_axis_name="c", subcore_axis_name="s", num_cores=NC, num_subcores=NS
)
_SC_PARAMS = pltpu.CompilerParams(use_tc_tiling_on_sc=False, needs_layout_passes=False)


def _mlp_body(x_ref, w1_ref, b1_ref, w2_ref, b2_ref, o_ref):
    a = jnp.dot(x_ref[...], w1_ref[...], preferred_element_type=jnp.float32)
    a = jnp.maximum(a + b1_ref[...], 0.0)
    o_ref[...] = (
        jnp.dot(a, w2_ref[...], preferred_element_type=jnp.float32) + b2_ref[...]
    )


def _mlp(x, W1, b1r, W2p, b2p):
    blk = 1024
    return pl.pallas_call(
        _mlp_body,
        grid=(NP // blk,),
        in_specs=[
            pl.BlockSpec((blk, D), lambda i: (i, 0)),
            pl.BlockSpec((D, H), lambda i: (0, 0)),
            pl.BlockSpec((1, H), lambda i: (0, 0)),
            pl.BlockSpec((H, CP), lambda i: (0, 0)),
            pl.BlockSpec((1, CP), lambda i: (0, 0)),
        ],
        out_specs=pl.BlockSpec((blk, CP), lambda i: (i, 0)),
        out_shape=jax.ShapeDtypeStruct((NP, CP), jnp.float32),
    )(x, W1, b1r, W2p, b2p)


@functools.partial(
    pl.kernel,
    out_type=jax.ShapeDtypeStruct((NC * NP, 16), jnp.float32),
    mesh=_MESH,
    compiler_params=_SC_PARAMS,
    scratch_types=[
        pltpu.VMEM((NB, BS), jnp.int32),
        pltpu.VMEM((NB, BS), jnp.float32),
        pltpu.VMEM((BS, 16), jnp.float32),
        pltpu.VMEM((ZR, 16), jnp.float32),
        pltpu.VMEM((NSL, 16), jnp.float32),
        pltpu.VMEM_SHARED((NP, 16), jnp.float32),
    ],
)
def _deg_kernel(colg, wg, degp, col_v, w_v, srows, zbuf, obuf, deg_sh):
    cid = lax.axis_index("c")
    sid = lax.axis_index("s")
    wid = cid * NS + sid
    pltpu.sync_copy(colg.at[wid], col_v)
    pltpu.sync_copy(wg.at[wid], w_v)

    @pl.loop(0, ZR)
    def _z(i):
        zbuf[i, :] = jnp.zeros((16,), jnp.float32)

    @pl.loop(0, NSL // ZR)
    def _zc(j):
        pltpu.sync_copy(zbuf, deg_sh.at[pl.ds(sid * NSL + j * ZR, ZR)])

    plsc.subcore_barrier()

    @pl.loop(0, NB)
    def _b(b):
        @pl.loop(0, BS // 16)
        def _g(g):
            w16 = w_v[b, pl.ds(g * 16, 16)]
            for l in range(16):
                srows[g * 16 + l, :] = jnp.full((16,), w16[l], jnp.float32)

        pltpu.sync_copy(srows, deg_sh.at[col_v.at[b]], add=True)

    plsc.subcore_barrier()
    pltpu.sync_copy(deg_sh.at[pl.ds(sid * NSL, NSL)], obuf)
    pltpu.sync_copy(obuf, degp.at[pl.ds(cid * NP + sid * NSL, NSL)])


def _dinv_body(degp_ref, o_ref):
    d = jnp.sum(degp_ref[0] + degp_ref[1], axis=1) * (1.0 / 16.0)
    o_ref[...] = jnp.where(d > 0, lax.rsqrt(jnp.where(d > 0, d, 1.0)), 0.0)


def _dinv(degp):
    return pl.pallas_call(
        _dinv_body,
        in_specs=[pl.BlockSpec((NC, NP, 16), lambda: (0, 0, 0))],
        out_specs=pl.BlockSpec((NP,), lambda: (0,)),
        out_shape=jax.ShapeDtypeStruct((NP,), jnp.float32),
    )(degp)


@functools.partial(
    pl.kernel,
    out_type=jax.ShapeDtypeStruct((NW, NB, BS), jnp.float32),
    mesh=_MESH,
    compiler_params=_SC_PARAMS,
    scratch_types=[
        pltpu.VMEM((NP,), jnp.float32),
        pltpu.VMEM((NB, BS), jnp.int32),
        pltpu.VMEM((NB, BS), jnp.int32),
        pltpu.VMEM((NB, BS), jnp.float32),
        pltpu.VMEM((NB, BS), jnp.float32),
    ],
)
def _norm_kernel(dinv, rowg, colg, wg, normg, dinv_v, row_v, col_v, w_v, norm_v):
    cid = lax.axis_index("c")
    sid = lax.axis_index("s")
    wid = cid * NS + sid
    pltpu.sync_copy(dinv, dinv_v)
    pltpu.sync_copy(rowg.at[wid], row_v)
    pltpu.sync_copy(colg.at[wid], col_v)
    pltpu.sync_copy(wg.at[wid], w_v)

    @pl.loop(0, NB)
    def _b(b):
        @pl.loop(0, BS // 16)
        def _g(g):
            sl = pl.ds(g * 16, 16)
            r16 = row_v[b, sl]
            c16 = col_v[b, sl]
            w16 = w_v[b, sl]
            dr = plsc.load_gather(dinv_v, [r16])
            dc = plsc.load_gather(dinv_v, [c16])
            norm_v[b, sl] = dr * w16 * dc

    pltpu.sync_copy(norm_v, normg.at[wid])


@functools.partial(
    pl.kernel,
    out_type=jax.ShapeDtypeStruct((NC * NP, CP), jnp.float32),
    mesh=_MESH,
    compiler_params=_SC_PARAMS,
    scratch_types=[
        pltpu.VMEM((NB, BS), jnp.int32),
        pltpu.VMEM((NB, BS), jnp.int32),
        pltpu.VMEM((NB, BS), jnp.float32),
        pltpu.VMEM((BSB, CP), jnp.float32),
        pltpu.VMEM((BSB, CP), jnp.float32),
        pltpu.VMEM((ZR, CP), jnp.float32),
        pltpu.VMEM((NSL // 2, CP), jnp.float32),
        pltpu.VMEM_SHARED((NP, CP), jnp.float32),
        pltpu.SemaphoreType.DMA,
        pltpu.SemaphoreType.DMA,
        pltpu.SemaphoreType.DMA,
        pltpu.SemaphoreType.DMA,
    ],
)
def _iter_kernel(
    out_hbm, rowg, colg, normg, p_out,
    row_v, col_v, norm_v, rows0, rows1, zbuf, obuf, agg_sh,
    gsem0, gsem1, ssem0, ssem1,
):
    cid = lax.axis_index("c")
    sid = lax.axis_index("s")
    wid = cid * NS + sid
    pltpu.sync_copy(rowg.at[wid], row_v)
    pltpu.sync_copy(colg.at[wid], col_v)
    pltpu.sync_copy(normg.at[wid], norm_v)

    @pl.loop(0, ZR)
    def _z(i):
        for k in range(CP // 16):
            zbuf[i, pl.ds(k * 16, 16)] = jnp.zeros((16,), jnp.float32)

    @pl.loop(0, NSL // ZR)
    def _zc(j):
        pltpu.sync_copy(zbuf, agg_sh.at[pl.ds(sid * NSL + j * ZR, ZR)])

    plsc.subcore_barrier()

    def _fire_gather(b, rows_v, sem):
        for j in range(GB):
            pltpu.async_copy(
                out_hbm.at[row_v.at[GB * b + j]],
                rows_v.at[pl.ds(j * BS, BS)],
                sem,
            )

    def _drain_gather(b, rows_v, sem):
        for j in range(GB):
            pltpu.make_async_copy(
                out_hbm.at[row_v.at[GB * b + j]],
                rows_v.at[pl.ds(j * BS, BS)],
                sem,
            ).wait()

    def _fire_scatter(b, rows_v, sem):
        for j in range(GB):
            pltpu.async_copy(
                rows_v.at[pl.ds(j * BS, BS)],
                agg_sh.at[col_v.at[GB * b + j]],
                sem,
                add=True,
            )

    def _drain_scatter(b, rows_v, sem):
        for j in range(GB):
            pltpu.make_async_copy(
                rows_v.at[pl.ds(j * BS, BS)],
                agg_sh.at[col_v.at[GB * b + j]],
                sem,
            ).wait()

    def _scale(b, rows_v):
        @pl.loop(0, GB)
        def _j(j):
            @pl.loop(0, BS // 16)
            def _g(g):
                n16 = norm_v[GB * b + j, pl.ds(g * 16, 16)]
                for l in range(16):
                    e = j * BS + g * 16 + l
                    nbv = jnp.full((16,), n16[l], jnp.float32)
                    for k in range(CP // 16):
                        sl = pl.ds(k * 16, 16)
                        rows_v[e, sl] = rows_v[e, sl] * nbv

    _fire_gather(0, rows0, gsem0)

    @pl.loop(0, NBB, step=2)
    def _b(b0):
        _drain_gather(b0, rows0, gsem0)

        @pl.when(b0 > 0)
        def _():
            _drain_scatter(b0 - 1, rows1, ssem1)

        _fire_gather(b0 + 1, rows1, gsem1)
        _scale(b0, rows0)
        _fire_scatter(b0, rows0, ssem0)

        _drain_gather(b0 + 1, rows1, gsem1)
        _drain_scatter(b0, rows0, ssem0)

        @pl.when(b0 + 2 < NBB)
        def _():
            _fire_gather(b0 + 2, rows0, gsem0)

        _scale(b0 + 1, rows1)
        _fire_scatter(b0 + 1, rows1, ssem1)

    _drain_scatter(NBB - 1, rows1, ssem1)
    plsc.subcore_barrier()

    @pl.loop(0, 2)
    def _o(c):
        off = sid * NSL + c * (NSL // 2)
        pltpu.sync_copy(agg_sh.at[pl.ds(off, NSL // 2)], obuf)
        pltpu.sync_copy(obuf, p_out.at[pl.ds(cid * NP + off, NSL // 2)])


WR = NP // NW


@functools.partial(
    pl.kernel,
    out_type=jax.ShapeDtypeStruct((NP, CP), jnp.float32),
    mesh=_MESH,
    compiler_params=_SC_PARAMS,
    scratch_types=[
        pltpu.VMEM((WR, CP), jnp.float32),
        pltpu.VMEM((WR, CP), jnp.float32),
        pltpu.VMEM((WR, CP), jnp.float32),
    ],
)
def _comb_kernel(p, h, out, b0, b1, hb):
    cid = lax.axis_index("c")
    sid = lax.axis_index("s")
    wid = cid * NS + sid
    base = wid * WR
    pltpu.sync_copy(p.at[pl.ds(base, WR)], b0)
    pltpu.sync_copy(p.at[pl.ds(NP + base, WR)], b1)
    pltpu.sync_copy(h.at[pl.ds(base, WR)], hb)

    @pl.loop(0, WR)
    def _i(i):
        for k in range(CP // 16):
            sl = pl.ds(k * 16, 16)
            b0[i, sl] = (1.0 - ALPHA) * (b0[i, sl] + b1[i, sl]) + ALPHA * hb[i, sl]

    pltpu.sync_copy(b0, out.at[pl.ds(base, WR)])


def _comb_body(p_ref, h_ref, o_ref):
    pb = p_ref[...]
    o_ref[...] = (1.0 - ALPHA) * (pb[0] + pb[1]) + ALPHA * h_ref[...]


def _combine(p, h):
    blk = 1024
    return pl.pallas_call(
        _comb_body,
        grid=(NP // blk,),
        in_specs=[
            pl.BlockSpec((NC, blk, CP), lambda i: (0, i, 0)),
            pl.BlockSpec((blk, CP), lambda i: (i, 0)),
        ],
        out_specs=pl.BlockSpec((blk, CP), lambda i: (i, 0)),
        out_shape=jax.ShapeDtypeStruct((NP, CP), jnp.float32),
    )(p, h)


def _final_body(p_ref, h_ref, o_ref):
    pb = p_ref[...]
    o = (1.0 - ALPHA) * (pb[0] + pb[1]) + ALPHA * h_ref[...]
    colid = lax.broadcasted_iota(jnp.int32, o.shape, 1)
    valid = colid < C
    om = jnp.where(valid, o, jnp.float32(-1e30))
    m = jnp.max(om, axis=1, keepdims=True)
    ex = jnp.where(valid, jnp.exp(o - m), 0.0)
    s = jnp.sum(ex, axis=1, keepdims=True)
    o_ref[...] = o - m - jnp.log(s)


def _final(p, h):
    blk = 1024
    return pl.pallas_call(
        _final_body,
        grid=(NP // blk,),
        in_specs=[
            pl.BlockSpec((NC, blk, CP), lambda i: (0, i, 0)),
            pl.BlockSpec((blk, CP), lambda i: (i, 0)),
        ],
        out_specs=pl.BlockSpec((blk, CP), lambda i: (i, 0)),
        out_shape=jax.ShapeDtypeStruct((NP, CP), jnp.float32),
    )(p, h)


def kernel(x, edge_index, edge_weight, W1, b1, W2, b2):
    row = edge_index[0].astype(jnp.int32)
    col = edge_index[1].astype(jnp.int32)
    loop = jnp.arange(N, dtype=jnp.int32)
    row = jnp.concatenate([row, loop])
    col = jnp.concatenate([col, loop])
    w = jnp.concatenate([edge_weight.astype(jnp.float32), jnp.ones((N,), jnp.float32)])

    pad = ET_PAD - row.shape[0]
    pidx = (jnp.arange(pad, dtype=jnp.int32) * 31) % N
    row_g = jnp.concatenate([row, pidx]).reshape(NW, NB, BS)
    col_g = jnp.concatenate([col, pidx]).reshape(NW, NB, BS)
    w_g = jnp.concatenate([w, jnp.zeros((pad,), jnp.float32)]).reshape(NW, NB, BS)

    W2p = jnp.pad(W2, ((0, 0), (0, CP - C)))
    b2p = jnp.pad(b2, (0, CP - C)).reshape(1, CP)
    b1r = b1.reshape(1, H)

    xp = jnp.pad(x, ((0, NP - N), (0, 0)))
    h = _mlp(xp, W1, b1r, W2p, b2p)
    degp = _deg_kernel(col_g, w_g).reshape(NC, NP, 16)
    dinv = _dinv(degp)
    norm_g = _norm_kernel(dinv, row_g, col_g, w_g)

    out = h
    for _ in range(K - 1):
        p = _iter_kernel(out, row_g, col_g, norm_g)
        out = _comb_kernel(p, h)
    p = _iter_kernel(out, row_g, col_g, norm_g).reshape(NC, NP, CP)
    res = _final(p, h)
    return res[:N, :C]


# speedup vs baseline: 32.2196x; 1.0163x over previous
import functools

import jax
import jax.numpy as jnp
from jax import lax
from jax.experimental import pallas as pl
from jax.experimental.pallas import tpu as pltpu
from jax.experimental.pallas import tpu_sc as plsc

N = 10000
D = 128
H = 32
C = 40
CP = 48
K = 10
ALPHA = 0.1

NC = 2
NS = 16
NW = NC * NS
BS = 128
NB = 84
GB = 2
NBB = NB // GB
BSB = GB * BS
PT = NB * BS
ET_PAD = NW * PT
NP = 10240
NSL = NP // NS
ZR = 128

_MESH = plsc.VectorSubcoreMesh(
    core_axis_name="c", subcore_axis_name="s", num_cores=NC, num_subcores=NS
)
_SC_PARAMS = pltpu.CompilerParams(use_tc_tiling_on_sc=False, needs_layout_passes=False)


def _mlp_body(x_ref, w1_ref, b1_ref, w2_ref, b2_ref, o_ref):
    a = jnp.dot(x_ref[...], w1_ref[...], preferred_element_type=jnp.float32)
    a = jnp.maximum(a + b1_ref[...], 0.0)
    o_ref[...] = (
        jnp.dot(a, w2_ref[...], preferred_element_type=jnp.float32) + b2_ref[...]
    )


def _mlp(x, W1, b1r, W2p, b2p):
    blk = 1024
    return pl.pallas_call(
        _mlp_body,
        grid=(NP // blk,),
        in_specs=[
            pl.BlockSpec((blk, D), lambda i: (i, 0)),
            pl.BlockSpec((D, H), lambda i: (0, 0)),
            pl.BlockSpec((1, H), lambda i: (0, 0)),
            pl.BlockSpec((H, CP), lambda i: (0, 0)),
            pl.BlockSpec((1, CP), lambda i: (0, 0)),
        ],
        out_specs=pl.BlockSpec((blk, CP), lambda i: (i, 0)),
        out_shape=jax.ShapeDtypeStruct((NP, CP), jnp.float32),
    )(x, W1, b1r, W2p, b2p)


@functools.partial(
    pl.kernel,
    out_type=jax.ShapeDtypeStruct((NC * NP, 16), jnp.float32),
    mesh=_MESH,
    compiler_params=_SC_PARAMS,
    scratch_types=[
        pltpu.VMEM((NB, BS), jnp.int32),
        pltpu.VMEM((NB, BS), jnp.float32),
        pltpu.VMEM((BS, 16), jnp.float32),
        pltpu.VMEM((ZR, 16), jnp.float32),
        pltpu.VMEM((NSL, 16), jnp.float32),
        pltpu.VMEM_SHARED((NP, 16), jnp.float32),
    ],
)
def _deg_kernel(colg, wg, degp, col_v, w_v, srows, zbuf, obuf, deg_sh):
    cid = lax.axis_index("c")
    sid = lax.axis_index("s")
    wid = cid * NS + sid
    pltpu.sync_copy(colg.at[wid], col_v)
    pltpu.sync_copy(wg.at[wid], w_v)

    @pl.loop(0, ZR)
    def _z(i):
        zbuf[i, :] = jnp.zeros((16,), jnp.float32)

    @pl.loop(0, NSL // ZR)
    def _zc(j):
        pltpu.sync_copy(zbuf, deg_sh.at[pl.ds(sid * NSL + j * ZR, ZR)])

    plsc.subcore_barrier()

    @pl.loop(0, NB)
    def _b(b):
        @pl.loop(0, BS // 16)
        def _g(g):
            w16 = w_v[b, pl.ds(g * 16, 16)]
            for l in range(16):
                srows[g * 16 + l, :] = jnp.full((16,), w16[l], jnp.float32)

        pltpu.sync_copy(srows, deg_sh.at[col_v.at[b]], add=True)

    plsc.subcore_barrier()
    pltpu.sync_copy(deg_sh.at[pl.ds(sid * NSL, NSL)], obuf)
    pltpu.sync_copy(obuf, degp.at[pl.ds(cid * NP + sid * NSL, NSL)])


def _dinv_body(degp_ref, o_ref):
    d = jnp.sum(degp_ref[0] + degp_ref[1], axis=1) * (1.0 / 16.0)
    o_ref[...] = jnp.where(d > 0, lax.rsqrt(jnp.where(d > 0, d, 1.0)), 0.0)


def _dinv(degp):
    return pl.pallas_call(
        _dinv_body,
        in_specs=[pl.BlockSpec((NC, NP, 16), lambda: (0, 0, 0))],
        out_specs=pl.BlockSpec((NP,), lambda: (0,)),
        out_shape=jax.ShapeDtypeStruct((NP,), jnp.float32),
    )(degp)


@functools.partial(
    pl.kernel,
    out_type=jax.ShapeDtypeStruct((NW, NB, BS), jnp.float32),
    mesh=_MESH,
    compiler_params=_SC_PARAMS,
    scratch_types=[
        pltpu.VMEM((NP,), jnp.float32),
        pltpu.VMEM((NB, BS), jnp.int32),
        pltpu.VMEM((NB, BS), jnp.int32),
        pltpu.VMEM((NB, BS), jnp.float32),
        pltpu.VMEM((NB, BS), jnp.float32),
    ],
)
def _norm_kernel(dinv, rowg, colg, wg, normg, dinv_v, row_v, col_v, w_v, norm_v):
    cid = lax.axis_index("c")
    sid = lax.axis_index("s")
    wid = cid * NS + sid
    pltpu.sync_copy(dinv, dinv_v)
    pltpu.sync_copy(rowg.at[wid], row_v)
    pltpu.sync_copy(colg.at[wid], col_v)
    pltpu.sync_copy(wg.at[wid], w_v)

    @pl.loop(0, NB)
    def _b(b):
        @pl.loop(0, BS // 16)
        def _g(g):
            sl = pl.ds(g * 16, 16)
            r16 = row_v[b, sl]
            c16 = col_v[b, sl]
            w16 = w_v[b, sl]
            dr = plsc.load_gather(dinv_v, [r16])
            dc = plsc.load_gather(dinv_v, [c16])
            norm_v[b, sl] = dr * w16 * dc

    pltpu.sync_copy(norm_v, normg.at[wid])


@functools.partial(
    pl.kernel,
    out_type=jax.ShapeDtypeStruct((NC * NP, CP), jnp.float32),
    mesh=_MESH,
    compiler_params=_SC_PARAMS,
    scratch_types=[
        pltpu.VMEM((NB, BS), jnp.int32),
        pltpu.VMEM((NB, BS), jnp.int32),
        pltpu.VMEM((NB, BS), jnp.float32),
        pltpu.VMEM((BSB, CP), jnp.float32),
        pltpu.VMEM((BSB, CP), jnp.float32),
        pltpu.VMEM((BSB, CP), jnp.float32),
        pltpu.VMEM((ZR, CP), jnp.float32),
        pltpu.VMEM_SHARED((NP, CP), jnp.float32),
        pltpu.SemaphoreType.DMA,
        pltpu.SemaphoreType.DMA,
        pltpu.SemaphoreType.DMA,
        pltpu.SemaphoreType.DMA,
        pltpu.SemaphoreType.DMA,
        pltpu.SemaphoreType.DMA,
    ],
)
def _iter_kernel(
    out_hbm, rowg, colg, normg, p_out,
    row_v, col_v, norm_v, rows0, rows1, rows2, zbuf, agg_sh,
    gsem0, gsem1, gsem2, ssem0, ssem1, ssem2,
):
    cid = lax.axis_index("c")
    sid = lax.axis_index("s")
    wid = cid * NS + sid
    pltpu.sync_copy(rowg.at[wid], row_v)
    pltpu.sync_copy(colg.at[wid], col_v)
    pltpu.sync_copy(normg.at[wid], norm_v)

    @pl.loop(0, ZR)
    def _z(i):
        for k in range(CP // 16):
            zbuf[i, pl.ds(k * 16, 16)] = jnp.zeros((16,), jnp.float32)

    @pl.loop(0, NSL // ZR)
    def _zc(j):
        pltpu.sync_copy(zbuf, agg_sh.at[pl.ds(sid * NSL + j * ZR, ZR)])

    plsc.subcore_barrier()

    bufs = (rows0, rows1, rows2)
    gsems = (gsem0, gsem1, gsem2)
    ssems = (ssem0, ssem1, ssem2)

    def _fire_gather(b, rows_v, sem):
        for j in range(GB):
            pltpu.async_copy(
                out_hbm.at[row_v.at[GB * b + j]],
                rows_v.at[pl.ds(j * BS, BS)],
                sem,
            )

    def _drain_gather(b, rows_v, sem):
        for j in range(GB):
            pltpu.make_async_copy(
                out_hbm.at[row_v.at[GB * b + j]],
                rows_v.at[pl.ds(j * BS, BS)],
                sem,
            ).wait()

    def _fire_scatter(b, rows_v, sem):
        for j in range(GB):
            pltpu.async_copy(
                rows_v.at[pl.ds(j * BS, BS)],
                agg_sh.at[col_v.at[GB * b + j]],
                sem,
                add=True,
            )

    def _drain_scatter(b, rows_v, sem):
        for j in range(GB):
            pltpu.make_async_copy(
                rows_v.at[pl.ds(j * BS, BS)],
                agg_sh.at[col_v.at[GB * b + j]],
                sem,
            ).wait()

    def _scale(b, rows_v):
        @pl.loop(0, GB)
        def _j(j):
            @pl.loop(0, BS // 16)
            def _g(g):
                n16 = norm_v[GB * b + j, pl.ds(g * 16, 16)]
                for l in range(16):
                    e = j * BS + g * 16 + l
                    nbv = jnp.full((16,), n16[l], jnp.float32)
                    for k in range(CP // 16):
                        sl = pl.ds(k * 16, 16)
                        rows_v[e, sl] = rows_v[e, sl] * nbv

    _fire_gather(0, rows0, gsem0)
    _fire_gather(1, rows1, gsem1)

    @pl.loop(0, NBB, step=3)
    def _b(b0):
        for st in range(3):
            b = b0 + st
            x = st
            z = (st + 2) % 3
            _drain_gather(b, bufs[x], gsems[x])

            @pl.when(b > 0)
            def _():
                _drain_scatter(b - 1, bufs[z], ssems[z])

            @pl.when(b + 2 < NBB)
            def _():
                _fire_gather(b + 2, bufs[z], gsems[z])

            _scale(b, bufs[x])
            _fire_scatter(b, bufs[x], ssems[x])

    _drain_scatter(NBB - 1, bufs[(NBB - 1) % 3], ssems[(NBB - 1) % 3])
    plsc.subcore_barrier()

    @pl.loop(0, NSL // ZR)
    def _o(c):
        off = sid * NSL + c * ZR
        pltpu.sync_copy(agg_sh.at[pl.ds(off, ZR)], zbuf)
        pltpu.sync_copy(zbuf, p_out.at[pl.ds(cid * NP + off, ZR)])


WR = NP // NW


@functools.partial(
    pl.kernel,
    out_type=jax.ShapeDtypeStruct((NP, CP), jnp.float32),
    mesh=_MESH,
    compiler_params=_SC_PARAMS,
    scratch_types=[
        pltpu.VMEM((WR, CP), jnp.float32),
        pltpu.VMEM((WR, CP), jnp.float32),
        pltpu.VMEM((WR, CP), jnp.float32),
    ],
)
def _comb_kernel(p, h, out, b0, b1, hb):
    cid = lax.axis_index("c")
    sid = lax.axis_index("s")
    wid = cid * NS + sid
    base = wid * WR
    pltpu.sync_copy(p.at[pl.ds(base, WR)], b0)
    pltpu.sync_copy(p.at[pl.ds(NP + base, WR)], b1)
    pltpu.sync_copy(h.at[pl.ds(base, WR)], hb)

    @pl.loop(0, WR)
    def _i(i):
        for k in range(CP // 16):
            sl = pl.ds(k * 16, 16)
            b0[i, sl] = (1.0 - ALPHA) * (b0[i, sl] + b1[i, sl]) + ALPHA * hb[i, sl]

    pltpu.sync_copy(b0, out.at[pl.ds(base, WR)])


def _comb_body(p_ref, h_ref, o_ref):
    pb = p_ref[...]
    o_ref[...] = (1.0 - ALPHA) * (pb[0] + pb[1]) + ALPHA * h_ref[...]


def _combine(p, h):
    blk = 1024
    return pl.pallas_call(
        _comb_body,
        grid=(NP // blk,),
        in_specs=[
            pl.BlockSpec((NC, blk, CP), lambda i: (0, i, 0)),
            pl.BlockSpec((blk, CP), lambda i: (i, 0)),
        ],
        out_specs=pl.BlockSpec((blk, CP), lambda i: (i, 0)),
        out_shape=jax.ShapeDtypeStruct((NP, CP), jnp.float32),
    )(p, h)


def _final_body(p_ref, h_ref, o_ref):
    pb = p_ref[...]
    o = (1.0 - ALPHA) * (pb[0] + pb[1]) + ALPHA * h_ref[...]
    colid = lax.broadcasted_iota(jnp.int32, o.shape, 1)
    valid = colid < C
    om = jnp.where(valid, o, jnp.float32(-1e30))
    m = jnp.max(om, axis=1, keepdims=True)
    ex = jnp.where(valid, jnp.exp(o - m), 0.0)
    s = jnp.sum(ex, axis=1, keepdims=True)
    o_ref[...] = o - m - jnp.log(s)


def _final(p, h):
    blk = 1024
    return pl.pallas_call(
        _final_body,
        grid=(NP // blk,),
        in_specs=[
            pl.BlockSpec((NC, blk, CP), lambda i: (0, i, 0)),
            pl.BlockSpec((blk, CP), lambda i: (i, 0)),
        ],
        out_specs=pl.BlockSpec((blk, CP), lambda i: (i, 0)),
        out_shape=jax.ShapeDtypeStruct((NP, CP), jnp.float32),
    )(p, h)


def kernel(x, edge_index, edge_weight, W1, b1, W2, b2):
    row = edge_index[0].astype(jnp.int32)
    col = edge_index[1].astype(jnp.int32)
    loop = jnp.arange(N, dtype=jnp.int32)
    row = jnp.concatenate([row, loop])
    col = jnp.concatenate([col, loop])
    w = jnp.concatenate([edge_weight.astype(jnp.float32), jnp.ones((N,), jnp.float32)])

    pad = ET_PAD - row.shape[0]
    pidx = (jnp.arange(pad, dtype=jnp.int32) * 31) % N
    row_g = jnp.concatenate([row, pidx]).reshape(NW, NB, BS)
    col_g = jnp.concatenate([col, pidx]).reshape(NW, NB, BS)
    w_g = jnp.concatenate([w, jnp.zeros((pad,), jnp.float32)]).reshape(NW, NB, BS)

    W2p = jnp.pad(W2, ((0, 0), (0, CP - C)))
    b2p = jnp.pad(b2, (0, CP - C)).reshape(1, CP)
    b1r = b1.reshape(1, H)

    xp = jnp.pad(x, ((0, NP - N), (0, 0)))
    h = _mlp(xp, W1, b1r, W2p, b2p)
    degp = _deg_kernel(col_g, w_g).reshape(NC, NP, 16)
    dinv = _dinv(degp)
    norm_g = _norm_kernel(dinv, row_g, col_g, w_g)

    out = h
    for _ in range(K - 1):
        p = _iter_kernel(out, row_g, col_g, norm_g)
        out = _comb_kernel(p, h)
    p = _iter_kernel(out, row_g, col_g, norm_g).reshape(NC, NP, CP)
    res = _final(p, h)
    return res[:N, :C]
